# Initial kernel scaffold; baseline (speedup 1.0000x reference)
import functools

import jax
import jax.numpy as jnp
from jax import lax
from jax.experimental import pallas as pl
from jax.experimental.pallas import tpu as pltpu
from jax.experimental.pallas import tpu_sc as plsc

NC = 2
NS = 16
L = 16
NW = NC * NS
CH = 128
R = 1000


def _deg_call(dst2d, n_acc):
  n_chunks = dst2d.shape[0] // NW
  mesh = plsc.VectorSubcoreMesh(core_axis_name="c", subcore_axis_name="s")

  @functools.partial(
      pl.kernel,
      out_type=jax.ShapeDtypeStruct((NW, n_acc), jnp.float32),
      mesh=mesh,
      scratch_types=[
          pltpu.VMEM((n_chunks, CH), jnp.int32),
          pltpu.VMEM((n_acc,), jnp.float32),
      ],
  )
  def deg_kernel(dst_hbm, out_hbm, didx, local):
    c = lax.axis_index("c")
    s = lax.axis_index("s")
    w = s * NC + c
    pltpu.sync_copy(dst_hbm.at[pl.ds(w * n_chunks, n_chunks)], didx)

    def zero_body(i, carry):
      local[pl.ds(i * L, L)] = jnp.zeros((L,), jnp.float32)
      return carry

    lax.fori_loop(0, n_acc // L, zero_body, 0)
    ones = jnp.ones((L,), jnp.float32)

    def chunk_body(j, carry):
      def lane_body(k, carry2):
        idx = didx[j, pl.ds(k * L, L)]
        plsc.addupdate_scatter(local, [idx], ones)
        return carry2

      lax.fori_loop(0, CH // L, lane_body, 0)
      return carry

    lax.fori_loop(0, n_chunks, chunk_body, 0)
    pltpu.sync_copy(local, out_hbm.at[w])

  return deg_kernel(dst2d)


def _agg_call(hp_flat, src2d, dst2d, n, n_acc, dh):
  n_chunks = src2d.shape[0] // NS
  rows_per_tec = n // NS
  slab = rows_per_tec // 5
  mesh = plsc.VectorSubcoreMesh(core_axis_name="c", subcore_axis_name="s")

  @functools.partial(
      pl.kernel,
      out_type=jax.ShapeDtypeStruct((NC * n, dh), jnp.float32),
      mesh=mesh,
      scratch_types=[
          pltpu.VMEM((n_chunks, CH), jnp.int32),
          pltpu.VMEM((n_chunks, CH), jnp.int32),
          pltpu.VMEM((CH, dh), jnp.float32),
          pltpu.VMEM_SHARED((n_acc, dh), jnp.float32),
          pltpu.SemaphoreType.DMA,
      ],
  )
  def agg_kernel(hp_hbm, src_hbm, dst_hbm, out_hbm, sidx, didx, buf, accum,
                 gsem):
    c = lax.axis_index("c")
    s = lax.axis_index("s")
    cn = c * n
    base = s * rows_per_tec

    pltpu.sync_copy(src_hbm.at[pl.ds(s * n_chunks, n_chunks)], sidx)
    pltpu.sync_copy(dst_hbm.at[pl.ds(s * n_chunks, n_chunks)], didx)

    cnv = jnp.full((L,), 1, jnp.int32) * cn

    def adj_chunk(j, carry):
      def adj_lane(k, carry2):
        sidx[j, pl.ds(k * L, L)] = sidx[j, pl.ds(k * L, L)] + cnv
        return carry2

      lax.fori_loop(0, CH // L, adj_lane, 0)
      return carry

    lax.fori_loop(0, n_chunks, adj_chunk, 0)

    def init_slab(k, carry):
      off = base + k * slab
      pltpu.sync_copy(hp_hbm.at[pl.ds(cn + off, slab)],
                      buf.at[pl.ds(0, slab)])
      pltpu.sync_copy(buf.at[pl.ds(0, slab)], accum.at[pl.ds(off, slab)])
      return carry

    lax.fori_loop(0, rows_per_tec // slab, init_slab, 0)
    plsc.subcore_barrier()

    def edge_chunk(j, carry):
      pltpu.async_copy(hp_hbm.at[sidx.at[j]], buf, gsem).wait()
      pltpu.sync_copy(buf, accum.at[didx.at[j]], add=True)
      return carry

    lax.fori_loop(0, n_chunks, edge_chunk, 0)
    plsc.subcore_barrier()

    def out_slab(k, carry):
      off = base + k * slab
      pltpu.sync_copy(accum.at[pl.ds(off, slab)], buf.at[pl.ds(0, slab)])
      pltpu.sync_copy(buf.at[pl.ds(0, slab)],
                      out_hbm.at[pl.ds(cn + off, slab)])
      return carry

    lax.fori_loop(0, rows_per_tec // slab, out_slab, 0)

  return agg_kernel(hp_flat, src2d, dst2d)


def _dis_col(dp):
  ones = jnp.ones((NW, 1), jnp.float32)
  deg = lax.dot_general(dp, ones, (((0,), (0,)), ((), ()))) + 1.0
  return lax.rsqrt(deg)


def _tc_pre(x, w1, degp):
  n, d_in = x.shape
  grid = n // R

  def body(x_ref, w_ref, dp_ref, out_ref):
    dis = _dis_col(dp_ref[...])
    h = jnp.dot(x_ref[...], w_ref[...])
    hp = h * dis
    out_ref[0] = hp[:, :128]
    out_ref[1] = hp[:, 128:]

  return pl.pallas_call(
      body,
      grid=(grid,),
      in_specs=[
          pl.BlockSpec((R, d_in), lambda i: (i, 0)),
          pl.BlockSpec(w1.shape, lambda i: (0, 0)),
          pl.BlockSpec((NW, R), lambda i: (0, i)),
      ],
      out_specs=pl.BlockSpec((2, R, 128), lambda i: (0, i, 0)),
      out_shape=jax.ShapeDtypeStruct((2, n, 128), jnp.float32),
  )(x, w1, degp)


def _tc_mid(a1, degp, w2, b1r):
  n = a1.shape[1]
  grid = n // R

  def body(a_ref, dp_ref, w_ref, b_ref, out_ref):
    dis = _dis_col(dp_ref[...])
    h0 = jnp.maximum(a_ref[0] * dis + b_ref[0:1, :128], 0.0)
    h1 = jnp.maximum(a_ref[1] * dis + b_ref[0:1, 128:], 0.0)
    h2 = jnp.dot(h0, w_ref[:128, :]) + jnp.dot(h1, w_ref[128:, :])
    hp2 = h2 * dis
    out_ref[0] = hp2[:, :64]
    out_ref[1] = hp2[:, 64:]

  return pl.pallas_call(
      body,
      grid=(grid,),
      in_specs=[
          pl.BlockSpec((2, R, 128), lambda i: (0, i, 0)),
          pl.BlockSpec((NW, R), lambda i: (0, i)),
          pl.BlockSpec(w2.shape, lambda i: (0, 0)),
          pl.BlockSpec(b1r.shape, lambda i: (0, 0)),
      ],
      out_specs=pl.BlockSpec((2, R, 64), lambda i: (0, i, 0)),
      out_shape=jax.ShapeDtypeStruct((2, n, 64), jnp.float32),
  )(a1, degp, w2, b1r)


def _tc_final(a2, degp, b2r, batch3, wl1p, bl1p, wl2p, bl2p, num_graphs):
  n = a2.shape[1]
  grid = n // R
  g = num_graphs

  def body(a_ref, dp_ref, b_ref, bat_ref, w1_ref, c1_ref, w2_ref, c2_ref,
           out_ref, psum, pcnt):
    i = pl.program_id(0)

    @pl.when(i == 0)
    def _init():
      psum[...] = jnp.zeros_like(psum)
      pcnt[...] = jnp.zeros_like(pcnt)

    dis = _dis_col(dp_ref[...])
    h3a = jnp.maximum(a_ref[0] * dis + b_ref[0:1, :64], 0.0)
    h3b = jnp.maximum(a_ref[1] * dis + b_ref[0:1, 64:], 0.0)
    h3 = jnp.concatenate([h3a, h3b], axis=1)

    seg = bat_ref[0]
    gids = lax.broadcasted_iota(jnp.int32, (g, 1), 0)
    oh = (seg == gids).astype(jnp.float32)
    dn = (((1,), (0,)), ((), ()))
    psum[...] += lax.dot_general(oh, h3, dn)
    pcnt[...] += lax.dot_general(oh, jnp.ones((R, 128), jnp.float32), dn)

    @pl.when(i == grid - 1)
    def _finish():
      gm = psum[...] / jnp.maximum(pcnt[...], 1.0)
      t = jnp.maximum(jnp.dot(gm, w1_ref[...]) + c1_ref[...], 0.0)
      z = jnp.dot(t, w2_ref[...]) + c2_ref[...]
      out_ref[...] = 1.0 / (1.0 + jnp.exp(-z))

  return pl.pallas_call(
      body,
      grid=(grid,),
      in_specs=[
          pl.BlockSpec((2, R, 64), lambda i: (0, i, 0)),
          pl.BlockSpec((NW, R), lambda i: (0, i)),
          pl.BlockSpec(b2r.shape, lambda i: (0, 0)),
          pl.BlockSpec((1, 1, R), lambda i: (i, 0, 0)),
          pl.BlockSpec(wl1p.shape, lambda i: (0, 0)),
          pl.BlockSpec(bl1p.shape, lambda i: (0, 0)),
          pl.BlockSpec(wl2p.shape, lambda i: (0, 0)),
          pl.BlockSpec(bl2p.shape, lambda i: (0, 0)),
      ],
      out_specs=pl.BlockSpec((g, 128), lambda i: (0, 0)),
      out_shape=jax.ShapeDtypeStruct((g, 128), jnp.float32),
      scratch_shapes=[
          pltpu.VMEM((g, 128), jnp.float32),
          pltpu.VMEM((g, 128), jnp.float32),
      ],
  )(a2, degp, b2r, batch3, wl1p, bl1p, wl2p, bl2p)


def kernel(x, edge_index, batch, W1, b1, W2, b2, Wl1, bl1, Wl2, bl2):
  n, d_in = x.shape
  e = edge_index.shape[1]
  num_graphs = 64

  ei = edge_index.astype(jnp.int32)
  src, dst = ei[0], ei[1]
  e_pad = ((e + NW * CH - 1) // (NW * CH)) * (NW * CH)
  src_p = jnp.concatenate([src, jnp.zeros((e_pad - e,), jnp.int32)])
  dst_p = jnp.concatenate([dst, jnp.full((e_pad - e,), n, jnp.int32)])
  src2d = src_p.reshape(-1, CH)
  dst2d = dst_p.reshape(-1, CH)
  n_acc = n + L

  degp_full = _deg_call(dst2d, n_acc)
  degp = degp_full[:, :n]

  hp1 = _tc_pre(x, W1, degp)
  a1 = _agg_call(hp1.reshape(2 * n, 128), src2d, dst2d, n, n_acc, 128)
  a1 = a1.reshape(2, n, 128)

  b1r = b1.reshape(1, 256)
  hp2 = _tc_mid(a1, degp, W2, b1r)
  a2 = _agg_call(hp2.reshape(2 * n, 64), src2d, dst2d, n, n_acc, 64)
  a2 = a2.reshape(2, n, 64)

  b2r = b2.reshape(1, 128)
  batch3 = batch.astype(jnp.int32).reshape(n // R, 1, R)
  wl1p = jnp.pad(Wl1, ((0, 0), (0, 64)))
  bl1p = jnp.pad(bl1, (0, 64)).reshape(1, 128)
  wl2p = jnp.pad(Wl2, ((0, 64), (0, 127)))
  bl2p = jnp.pad(bl2, (0, 127)).reshape(1, 128)

  outp = _tc_final(a2, degp, b2r, batch3, wl1p, bl1p, wl2p, bl2p,
                   num_graphs)
  return outp[:, :1]


# speedup vs baseline: 8.7119x; 8.7119x over previous
import functools

import jax
import jax.numpy as jnp
from jax import lax
from jax.experimental import pallas as pl
from jax.experimental.pallas import tpu as pltpu
from jax.experimental.pallas import tpu_sc as plsc

NC = 2
NS = 16
L = 16
NW = NC * NS
CH = 128
R = 1024


def _deg_call(dst2d, n_acc):
  n_chunks = dst2d.shape[0] // NW
  sl = n_acc // NS
  mesh = plsc.VectorSubcoreMesh(core_axis_name="c", subcore_axis_name="s")

  @functools.partial(
      pl.kernel,
      out_type=jax.ShapeDtypeStruct((NC * n_acc,), jnp.float32),
      mesh=mesh,
      scratch_types=[
          pltpu.VMEM((n_chunks, CH), jnp.int32),
          pltpu.VMEM((CH,), jnp.float32),
          pltpu.VMEM((n_acc,), jnp.float32),
          pltpu.VMEM_SHARED((n_acc,), jnp.float32),
      ],
  )
  def deg_kernel(dst_hbm, out_hbm, didx, onesb, bounce, accum):
    c = lax.axis_index("c")
    s = lax.axis_index("s")
    base = (c * NS + s) * n_chunks
    pltpu.sync_copy(dst_hbm.at[pl.ds(base, n_chunks)], didx)

    def fill(i, carry):
      bounce[pl.ds(i * L, L)] = jnp.zeros((L,), jnp.float32)
      return carry

    lax.fori_loop(0, n_acc // L, fill, 0)

    def fill1(i, carry):
      onesb[pl.ds(i * L, L)] = jnp.ones((L,), jnp.float32)
      return carry

    lax.fori_loop(0, CH // L, fill1, 0)
    pltpu.sync_copy(bounce.at[pl.ds(0, sl)], accum.at[pl.ds(s * sl, sl)])
    plsc.subcore_barrier()

    def chunk_body(j, carry):
      pltpu.sync_copy(onesb, accum.at[didx.at[j]], add=True)
      return carry

    lax.fori_loop(0, n_chunks, chunk_body, 0)
    plsc.subcore_barrier()

    @pl.when(s == 0)
    def _out():
      pltpu.sync_copy(accum, bounce)
      pltpu.sync_copy(bounce, out_hbm.at[pl.ds(c * n_acc, n_acc)])

  return deg_kernel(dst2d).reshape(NC, n_acc)


def _agg_call(hp_flat, src2d, dst2d, n, n_acc, dh):
  n_chunks = src2d.shape[0] // NS
  rows_per_tec = n // NS
  slab = CH
  ib = 32
  mesh = plsc.VectorSubcoreMesh(core_axis_name="c", subcore_axis_name="s")

  @functools.partial(
      pl.kernel,
      out_type=jax.ShapeDtypeStruct((NC * n, dh), jnp.float32),
      mesh=mesh,
      scratch_types=[
          pltpu.VMEM((ib, CH), jnp.int32),
          pltpu.VMEM((ib, CH), jnp.int32),
          pltpu.VMEM((CH, dh), jnp.float32),
          pltpu.VMEM_SHARED((n_acc, dh), jnp.float32),
          pltpu.SemaphoreType.DMA,
      ],
  )
  def agg_kernel(hp_hbm, src_hbm, dst_hbm, out_hbm, sidx, didx, buf, accum,
                 gsem):
    c = lax.axis_index("c")
    s = lax.axis_index("s")
    cn = c * n
    base = s * rows_per_tec
    cnv = jnp.full((L,), 1, jnp.int32) * cn

    def init_slab(k, carry):
      off = base + k * slab
      pltpu.sync_copy(hp_hbm.at[pl.ds(cn + off, slab)],
                      buf.at[pl.ds(0, slab)])
      pltpu.sync_copy(buf.at[pl.ds(0, slab)], accum.at[pl.ds(off, slab)])
      return carry

    lax.fori_loop(0, rows_per_tec // slab, init_slab, 0)
    plsc.subcore_barrier()

    def blk_body(t, carry):
      boff = s * n_chunks + t * ib
      pltpu.sync_copy(src_hbm.at[pl.ds(boff, ib)], sidx)
      pltpu.sync_copy(dst_hbm.at[pl.ds(boff, ib)], didx)

      def adj_chunk(j, carry2):
        def adj_lane(k, carry3):
          sidx[j, pl.ds(k * L, L)] = sidx[j, pl.ds(k * L, L)] + cnv
          return carry3

        lax.fori_loop(0, CH // L, adj_lane, 0)
        return carry2

      lax.fori_loop(0, ib, adj_chunk, 0)

      def edge_chunk(j, carry2):
        pltpu.async_copy(hp_hbm.at[sidx.at[j]], buf, gsem).wait()
        pltpu.sync_copy(buf, accum.at[didx.at[j]], add=True)
        return carry2

      lax.fori_loop(0, ib, edge_chunk, 0)
      return carry

    lax.fori_loop(0, n_chunks // ib, blk_body, 0)
    plsc.subcore_barrier()

    def out_slab(k, carry):
      off = base + k * slab
      pltpu.sync_copy(accum.at[pl.ds(off, slab)], buf.at[pl.ds(0, slab)])
      pltpu.sync_copy(buf.at[pl.ds(0, slab)],
                      out_hbm.at[pl.ds(cn + off, slab)])
      return carry

    lax.fori_loop(0, rows_per_tec // slab, out_slab, 0)

  return agg_kernel(hp_flat, src2d, dst2d)


def _agg2_call(hp, src2d, dst2d, n, n_acc):
  dh = hp.shape[1]
  n_chunks = src2d.shape[0] // NW
  rows_per_tec = n // NS
  slab = CH
  mesh = plsc.VectorSubcoreMesh(core_axis_name="c", subcore_axis_name="s")

  @functools.partial(
      pl.kernel,
      out_type=jax.ShapeDtypeStruct((NC * n, dh), jnp.float32),
      mesh=mesh,
      scratch_types=[
          pltpu.VMEM((n_chunks, CH), jnp.int32),
          pltpu.VMEM((n_chunks, CH), jnp.int32),
          pltpu.VMEM((CH, dh), jnp.float32),
          pltpu.VMEM_SHARED((n_acc, dh), jnp.float32),
          pltpu.SemaphoreType.DMA,
      ],
  )
  def agg2_kernel(hp_hbm, src_hbm, dst_hbm, out_hbm, sidx, didx, buf, accum,
                  gsem):
    c = lax.axis_index("c")
    s = lax.axis_index("s")
    base = s * rows_per_tec
    boff = (c * NS + s) * n_chunks

    pltpu.sync_copy(src_hbm.at[pl.ds(boff, n_chunks)], sidx)
    pltpu.sync_copy(dst_hbm.at[pl.ds(boff, n_chunks)], didx)

    def init_slab(k, carry):
      off = base + k * slab
      pltpu.sync_copy(hp_hbm.at[pl.ds(off, slab)], buf.at[pl.ds(0, slab)])
      pltpu.sync_copy(buf.at[pl.ds(0, slab)], accum.at[pl.ds(off, slab)])
      return carry

    lax.fori_loop(0, rows_per_tec // slab, init_slab, 0)
    plsc.subcore_barrier()

    def edge_chunk(j, carry):
      pltpu.async_copy(hp_hbm.at[sidx.at[j]], buf, gsem).wait()
      pltpu.sync_copy(buf, accum.at[didx.at[j]], add=True)
      return carry

    lax.fori_loop(0, n_chunks, edge_chunk, 0)
    plsc.subcore_barrier()

    def out_slab(k, carry):
      off = base + k * slab
      pltpu.sync_copy(accum.at[pl.ds(off, slab)], buf.at[pl.ds(0, slab)])
      pltpu.sync_copy(buf.at[pl.ds(0, slab)],
                      out_hbm.at[pl.ds(c * n + off, slab)])
      return carry

    lax.fori_loop(0, rows_per_tec // slab, out_slab, 0)

  return agg2_kernel(hp, src2d, dst2d)


def _dis_col(dp):
  ones = jnp.ones((dp.shape[0], 1), jnp.float32)
  deg = lax.dot_general(dp, ones, (((0,), (0,)), ((), ()))) + 1.0
  return lax.rsqrt(deg)


def _tc_pre(x, w1, degp):
  n, d_in = x.shape
  grid = n // R

  def body(x_ref, w_ref, dp_ref, out_ref):
    dis = _dis_col(dp_ref[...])
    h = jnp.dot(x_ref[...], w_ref[...])
    hp = h * dis
    out_ref[0] = hp[:, :128]
    out_ref[1] = hp[:, 128:]

  return pl.pallas_call(
      body,
      grid=(grid,),
      in_specs=[
          pl.BlockSpec((R, d_in), lambda i: (i, 0)),
          pl.BlockSpec(w1.shape, lambda i: (0, 0)),
          pl.BlockSpec((NC, R), lambda i: (0, i)),
      ],
      out_specs=pl.BlockSpec((2, R, 128), lambda i: (0, i, 0)),
      out_shape=jax.ShapeDtypeStruct((2, n, 128), jnp.float32),
  )(x, w1, degp)


def _tc_mid(a1, degp, w2, b1r):
  n = a1.shape[1]
  grid = n // R

  def body(a_ref, dp_ref, w_ref, b_ref, out_ref):
    dis = _dis_col(dp_ref[...])
    h0 = jnp.maximum(a_ref[0] * dis + b_ref[0:1, :128], 0.0)
    h1 = jnp.maximum(a_ref[1] * dis + b_ref[0:1, 128:], 0.0)
    h2 = jnp.dot(h0, w_ref[:128, :]) + jnp.dot(h1, w_ref[128:, :])
    out_ref[...] = h2 * dis

  return pl.pallas_call(
      body,
      grid=(grid,),
      in_specs=[
          pl.BlockSpec((2, R, 128), lambda i: (0, i, 0)),
          pl.BlockSpec((NC, R), lambda i: (0, i)),
          pl.BlockSpec(w2.shape, lambda i: (0, 0)),
          pl.BlockSpec(b1r.shape, lambda i: (0, 0)),
      ],
      out_specs=pl.BlockSpec((R, 128), lambda i: (i, 0)),
      out_shape=jax.ShapeDtypeStruct((n, 128), jnp.float32),
  )(a1, degp, w2, b1r)


def _tc_final(a2, hp2, degp, b2r, batch3, wl1p, bl1p, wl2p, bl2p, num_graphs):
  n = a2.shape[1]
  grid = n // R
  g = num_graphs

  def body(a_ref, hp_ref, dp_ref, b_ref, bat_ref, w1_ref, c1_ref, w2_ref,
           c2_ref, out_ref, psum, pcnt):
    i = pl.program_id(0)

    @pl.when(i == 0)
    def _init():
      psum[...] = jnp.zeros_like(psum)
      pcnt[...] = jnp.zeros_like(pcnt)

    dis = _dis_col(dp_ref[...])
    a = a_ref[0] + a_ref[1] - hp_ref[...]
    h3 = jnp.maximum(a * dis + b_ref[...], 0.0)

    seg = bat_ref[0]
    gids = lax.broadcasted_iota(jnp.int32, (g, 1), 0)
    oh = (seg == gids).astype(jnp.float32)
    dn = (((1,), (0,)), ((), ()))
    psum[...] += lax.dot_general(oh, h3, dn)
    pcnt[...] += lax.dot_general(oh, jnp.ones((R, 128), jnp.float32), dn)

    @pl.when(i == grid - 1)
    def _finish():
      gm = psum[...] / jnp.maximum(pcnt[...], 1.0)
      t = jnp.maximum(jnp.dot(gm, w1_ref[...]) + c1_ref[...], 0.0)
      z = jnp.dot(t, w2_ref[...]) + c2_ref[...]
      out_ref[...] = 1.0 / (1.0 + jnp.exp(-z))

  return pl.pallas_call(
      body,
      grid=(grid,),
      in_specs=[
          pl.BlockSpec((2, R, 128), lambda i: (0, i, 0)),
          pl.BlockSpec((R, 128), lambda i: (i, 0)),
          pl.BlockSpec((NC, R), lambda i: (0, i)),
          pl.BlockSpec(b2r.shape, lambda i: (0, 0)),
          pl.BlockSpec((1, 1, R), lambda i: (i, 0, 0)),
          pl.BlockSpec(wl1p.shape, lambda i: (0, 0)),
          pl.BlockSpec(bl1p.shape, lambda i: (0, 0)),
          pl.BlockSpec(wl2p.shape, lambda i: (0, 0)),
          pl.BlockSpec(bl2p.shape, lambda i: (0, 0)),
      ],
      out_specs=pl.BlockSpec((g, 128), lambda i: (0, 0)),
      out_shape=jax.ShapeDtypeStruct((g, 128), jnp.float32),
      scratch_shapes=[
          pltpu.VMEM((g, 128), jnp.float32),
          pltpu.VMEM((g, 128), jnp.float32),
      ],
  )(a2, hp2, degp, b2r, batch3, wl1p, bl1p, wl2p, bl2p)


def kernel(x, edge_index, batch, W1, b1, W2, b2, Wl1, bl1, Wl2, bl2):
  n0, d_in = x.shape
  e = edge_index.shape[1]
  num_graphs = 64

  n = ((n0 + R - 1) // R) * R
  x = jnp.pad(x, ((0, n - n0), (0, 0)))
  batch = jnp.pad(batch.astype(jnp.int32), (0, n - n0),
                  constant_values=num_graphs)

  ei = edge_index.astype(jnp.int32)
  src, dst = ei[0], ei[1]
  eq = NW * CH * 8
  e_pad = ((e + eq - 1) // eq) * eq
  src_p = jnp.concatenate([src, jnp.zeros((e_pad - e,), jnp.int32)])
  dst_p = jnp.concatenate([dst, jnp.full((e_pad - e,), n, jnp.int32)])
  src2d = src_p.reshape(-1, CH)
  dst2d = dst_p.reshape(-1, CH)
  n_acc = n + NS * L

  degp_full = _deg_call(dst2d, n_acc)
  degp = degp_full[:, :n]

  hp1 = _tc_pre(x, W1, degp)
  a1 = _agg_call(hp1.reshape(2 * n, 128), src2d, dst2d, n, n_acc, 128)
  a1 = a1.reshape(2, n, 128)

  b1r = b1.reshape(1, 256)
  hp2 = _tc_mid(a1, degp, W2, b1r)
  a2 = _agg2_call(hp2, src2d, dst2d, n, n_acc)
  a2 = a2.reshape(2, n, 128)

  b2r = b2.reshape(1, 128)
  batch3 = batch.reshape(n // R, 1, R)
  wl1p = jnp.pad(Wl1, ((0, 0), (0, 64)))
  bl1p = jnp.pad(bl1, (0, 64)).reshape(1, 128)
  wl2p = jnp.pad(Wl2, ((0, 64), (0, 127)))
  bl2p = jnp.pad(bl2, (0, 127)).reshape(1, 128)

  outp = _tc_final(a2, hp2, degp, b2r, batch3, wl1p, bl1p, wl2p, bl2p,
                   num_graphs)
  return outp[:, :1]


# speedup vs baseline: 9.6453x; 1.1071x over previous
import functools

import jax
import jax.numpy as jnp
from jax import lax
from jax.experimental import pallas as pl
from jax.experimental.pallas import tpu as pltpu
from jax.experimental.pallas import tpu_sc as plsc

NC = 2
NS = 16
L = 16
NW = NC * NS
CH = 128
R = 1024


def _deg_call(dst2d, n_acc):
  n_chunks = dst2d.shape[0] // NW
  sl = n_acc // NS
  mesh = plsc.VectorSubcoreMesh(core_axis_name="c", subcore_axis_name="s")

  @functools.partial(
      pl.kernel,
      out_type=jax.ShapeDtypeStruct((NC * n_acc,), jnp.float32),
      mesh=mesh,
      scratch_types=[
          pltpu.VMEM((n_chunks, CH), jnp.int32),
          pltpu.VMEM((CH,), jnp.float32),
          pltpu.VMEM((n_acc,), jnp.float32),
          pltpu.VMEM_SHARED((n_acc,), jnp.float32),
      ],
  )
  def deg_kernel(dst_hbm, out_hbm, didx, onesb, bounce, accum):
    c = lax.axis_index("c")
    s = lax.axis_index("s")
    base = (c * NS + s) * n_chunks
    pltpu.sync_copy(dst_hbm.at[pl.ds(base, n_chunks)], didx)

    def fill(i, carry):
      bounce[pl.ds(i * L, L)] = jnp.zeros((L,), jnp.float32)
      return carry

    lax.fori_loop(0, n_acc // L, fill, 0)

    def fill1(i, carry):
      onesb[pl.ds(i * L, L)] = jnp.ones((L,), jnp.float32)
      return carry

    lax.fori_loop(0, CH // L, fill1, 0)
    pltpu.sync_copy(bounce.at[pl.ds(0, sl)], accum.at[pl.ds(s * sl, sl)])
    plsc.subcore_barrier()

    def chunk_body(j, carry):
      pltpu.sync_copy(onesb, accum.at[didx.at[j]], add=True)
      return carry

    lax.fori_loop(0, n_chunks, chunk_body, 0)
    plsc.subcore_barrier()

    @pl.when(s == 0)
    def _out():
      pltpu.sync_copy(accum, bounce)
      pltpu.sync_copy(bounce, out_hbm.at[pl.ds(c * n_acc, n_acc)])

  return deg_kernel(dst2d).reshape(NC, n_acc)


def _edge_pipeline(hp_hbm, accum, sidx, didx, buf_a, buf_b, g_a, g_b, s_a,
                   s_b, nchunks):
  nt = nchunks // 2

  def pair(t, carry):
    j = 2 * t
    pltpu.make_async_copy(hp_hbm.at[sidx.at[j]], buf_a, g_a).wait()

    @pl.when(t > 0)
    def _wait_b():
      pltpu.make_async_copy(buf_b, accum.at[didx.at[0]], s_b).wait()

    pltpu.async_copy(hp_hbm.at[sidx.at[j + 1]], buf_b, g_b)
    pltpu.async_copy(buf_a, accum.at[didx.at[j]], s_a, add=True)
    pltpu.make_async_copy(hp_hbm.at[sidx.at[j + 1]], buf_b, g_b).wait()
    pltpu.make_async_copy(buf_a, accum.at[didx.at[0]], s_a).wait()

    @pl.when(t + 1 < nt)
    def _next_a():
      pltpu.async_copy(hp_hbm.at[sidx.at[j + 2]], buf_a, g_a)

    pltpu.async_copy(buf_b, accum.at[didx.at[j + 1]], s_b, add=True)
    return carry

  pltpu.async_copy(hp_hbm.at[sidx.at[0]], buf_a, g_a)
  lax.fori_loop(0, nt, pair, 0)
  pltpu.make_async_copy(buf_b, accum.at[didx.at[0]], s_b).wait()


def _agg_call(hp_flat, src2d, dst2d, n, n_acc, dh):
  n_chunks = src2d.shape[0] // NS
  rows_per_tec = n // NS
  slab = CH
  ib = 32
  mesh = plsc.VectorSubcoreMesh(core_axis_name="c", subcore_axis_name="s")

  @functools.partial(
      pl.kernel,
      out_type=jax.ShapeDtypeStruct((NC * n, dh), jnp.float32),
      mesh=mesh,
      scratch_types=[
          pltpu.VMEM((ib, CH), jnp.int32),
          pltpu.VMEM((ib, CH), jnp.int32),
          pltpu.VMEM((CH, dh), jnp.float32),
          pltpu.VMEM((CH, dh), jnp.float32),
          pltpu.VMEM_SHARED((n_acc, dh), jnp.float32),
          pltpu.SemaphoreType.DMA,
          pltpu.SemaphoreType.DMA,
          pltpu.SemaphoreType.DMA,
          pltpu.SemaphoreType.DMA,
      ],
  )
  def agg_kernel(hp_hbm, src_hbm, dst_hbm, out_hbm, sidx, didx, buf_a,
                 buf_b, accum, g_a, g_b, s_a, s_b):
    c = lax.axis_index("c")
    s = lax.axis_index("s")
    cn = c * n
    base = s * rows_per_tec
    cnv = jnp.full((L,), 1, jnp.int32) * cn

    def init_slab(k, carry):
      off = base + k * slab
      pltpu.sync_copy(hp_hbm.at[pl.ds(cn + off, slab)],
                      buf_a.at[pl.ds(0, slab)])
      pltpu.sync_copy(buf_a.at[pl.ds(0, slab)], accum.at[pl.ds(off, slab)])
      return carry

    lax.fori_loop(0, rows_per_tec // slab, init_slab, 0)
    plsc.subcore_barrier()

    def blk_body(t, carry):
      boff = s * n_chunks + t * ib
      pltpu.sync_copy(src_hbm.at[pl.ds(boff, ib)], sidx)
      pltpu.sync_copy(dst_hbm.at[pl.ds(boff, ib)], didx)

      def adj_chunk(j, carry2):
        def adj_lane(k, carry3):
          sidx[j, pl.ds(k * L, L)] = sidx[j, pl.ds(k * L, L)] + cnv
          return carry3

        lax.fori_loop(0, CH // L, adj_lane, 0)
        return carry2

      lax.fori_loop(0, ib, adj_chunk, 0)
      _edge_pipeline(hp_hbm, accum, sidx, didx, buf_a, buf_b, g_a, g_b,
                     s_a, s_b, ib)
      return carry

    lax.fori_loop(0, n_chunks // ib, blk_body, 0)
    plsc.subcore_barrier()

    def out_slab(k, carry):
      off = base + k * slab
      pltpu.sync_copy(accum.at[pl.ds(off, slab)], buf_a.at[pl.ds(0, slab)])
      pltpu.sync_copy(buf_a.at[pl.ds(0, slab)],
                      out_hbm.at[pl.ds(cn + off, slab)])
      return carry

    lax.fori_loop(0, rows_per_tec // slab, out_slab, 0)

  return agg_kernel(hp_flat, src2d, dst2d)


def _agg2_call(hp, src2d, dst2d, n, n_acc):
  dh = hp.shape[1]
  n_chunks = src2d.shape[0] // NW
  rows_per_tec = n // NS
  slab = CH
  ib = n_chunks // 2
  mesh = plsc.VectorSubcoreMesh(core_axis_name="c", subcore_axis_name="s")

  @functools.partial(
      pl.kernel,
      out_type=jax.ShapeDtypeStruct((NC * n, dh), jnp.float32),
      mesh=mesh,
      scratch_types=[
          pltpu.VMEM((ib, CH), jnp.int32),
          pltpu.VMEM((ib, CH), jnp.int32),
          pltpu.VMEM((CH, dh), jnp.float32),
          pltpu.VMEM((CH, dh), jnp.float32),
          pltpu.VMEM_SHARED((n_acc, dh), jnp.float32),
          pltpu.SemaphoreType.DMA,
          pltpu.SemaphoreType.DMA,
          pltpu.SemaphoreType.DMA,
          pltpu.SemaphoreType.DMA,
      ],
  )
  def agg2_kernel(hp_hbm, src_hbm, dst_hbm, out_hbm, sidx, didx, buf_a,
                  buf_b, accum, g_a, g_b, s_a, s_b):
    c = lax.axis_index("c")
    s = lax.axis_index("s")
    base = s * rows_per_tec

    def init_slab(k, carry):
      off = base + k * slab
      pltpu.sync_copy(hp_hbm.at[pl.ds(off, slab)], buf_a.at[pl.ds(0, slab)])
      pltpu.sync_copy(buf_a.at[pl.ds(0, slab)], accum.at[pl.ds(off, slab)])
      return carry

    lax.fori_loop(0, rows_per_tec // slab, init_slab, 0)
    plsc.subcore_barrier()

    def blk_body(t, carry):
      boff = (c * NS + s) * n_chunks + t * ib
      pltpu.sync_copy(src_hbm.at[pl.ds(boff, ib)], sidx)
      pltpu.sync_copy(dst_hbm.at[pl.ds(boff, ib)], didx)
      _edge_pipeline(hp_hbm, accum, sidx, didx, buf_a, buf_b, g_a, g_b,
                     s_a, s_b, ib)
      return carry

    lax.fori_loop(0, n_chunks // ib, blk_body, 0)
    plsc.subcore_barrier()

    def out_slab(k, carry):
      off = base + k * slab
      pltpu.sync_copy(accum.at[pl.ds(off, slab)], buf_a.at[pl.ds(0, slab)])
      pltpu.sync_copy(buf_a.at[pl.ds(0, slab)],
                      out_hbm.at[pl.ds(c * n + off, slab)])
      return carry

    lax.fori_loop(0, rows_per_tec // slab, out_slab, 0)

  return agg2_kernel(hp, src2d, dst2d)


def _dis_col(dp):
  ones = jnp.ones((dp.shape[0], 1), jnp.float32)
  deg = lax.dot_general(dp, ones, (((0,), (0,)), ((), ()))) + 1.0
  return lax.rsqrt(deg)


def _tc_pre(x, w1, degp):
  n, d_in = x.shape
  grid = n // R

  def body(x_ref, w_ref, dp_ref, out_ref):
    dis = _dis_col(dp_ref[...])
    h = jnp.dot(x_ref[...], w_ref[...])
    hp = h * dis
    out_ref[0] = hp[:, :128]
    out_ref[1] = hp[:, 128:]

  return pl.pallas_call(
      body,
      grid=(grid,),
      in_specs=[
          pl.BlockSpec((R, d_in), lambda i: (i, 0)),
          pl.BlockSpec(w1.shape, lambda i: (0, 0)),
          pl.BlockSpec((NC, R), lambda i: (0, i)),
      ],
      out_specs=pl.BlockSpec((2, R, 128), lambda i: (0, i, 0)),
      out_shape=jax.ShapeDtypeStruct((2, n, 128), jnp.float32),
  )(x, w1, degp)


def _tc_mid(a1, degp, w2, b1r):
  n = a1.shape[1]
  grid = n // R

  def body(a_ref, dp_ref, w_ref, b_ref, out_ref):
    dis = _dis_col(dp_ref[...])
    h0 = jnp.maximum(a_ref[0] * dis + b_ref[0:1, :128], 0.0)
    h1 = jnp.maximum(a_ref[1] * dis + b_ref[0:1, 128:], 0.0)
    h2 = jnp.dot(h0, w_ref[:128, :]) + jnp.dot(h1, w_ref[128:, :])
    out_ref[...] = h2 * dis

  return pl.pallas_call(
      body,
      grid=(grid,),
      in_specs=[
          pl.BlockSpec((2, R, 128), lambda i: (0, i, 0)),
          pl.BlockSpec((NC, R), lambda i: (0, i)),
          pl.BlockSpec(w2.shape, lambda i: (0, 0)),
          pl.BlockSpec(b1r.shape, lambda i: (0, 0)),
      ],
      out_specs=pl.BlockSpec((R, 128), lambda i: (i, 0)),
      out_shape=jax.ShapeDtypeStruct((n, 128), jnp.float32),
  )(a1, degp, w2, b1r)


def _tc_final(a2, hp2, degp, b2r, batch3, wl1p, bl1p, wl2p, bl2p, num_graphs):
  n = a2.shape[1]
  grid = n // R
  g = num_graphs

  def body(a_ref, hp_ref, dp_ref, b_ref, bat_ref, w1_ref, c1_ref, w2_ref,
           c2_ref, out_ref, psum, pcnt):
    i = pl.program_id(0)

    @pl.when(i == 0)
    def _init():
      psum[...] = jnp.zeros_like(psum)
      pcnt[...] = jnp.zeros_like(pcnt)

    dis = _dis_col(dp_ref[...])
    a = a_ref[0] + a_ref[1] - hp_ref[...]
    h3 = jnp.maximum(a * dis + b_ref[...], 0.0)

    seg = bat_ref[0]
    gids = lax.broadcasted_iota(jnp.int32, (g, 1), 0)
    oh = (seg == gids).astype(jnp.float32)
    dn = (((1,), (0,)), ((), ()))
    psum[...] += lax.dot_general(oh, h3, dn)
    pcnt[...] += lax.dot_general(oh, jnp.ones((R, 128), jnp.float32), dn)

    @pl.when(i == grid - 1)
    def _finish():
      gm = psum[...] / jnp.maximum(pcnt[...], 1.0)
      t = jnp.maximum(jnp.dot(gm, w1_ref[...]) + c1_ref[...], 0.0)
      z = jnp.dot(t, w2_ref[...]) + c2_ref[...]
      out_ref[...] = 1.0 / (1.0 + jnp.exp(-z))

  return pl.pallas_call(
      body,
      grid=(grid,),
      in_specs=[
          pl.BlockSpec((2, R, 128), lambda i: (0, i, 0)),
          pl.BlockSpec((R, 128), lambda i: (i, 0)),
          pl.BlockSpec((NC, R), lambda i: (0, i)),
          pl.BlockSpec(b2r.shape, lambda i: (0, 0)),
          pl.BlockSpec((1, 1, R), lambda i: (i, 0, 0)),
          pl.BlockSpec(wl1p.shape, lambda i: (0, 0)),
          pl.BlockSpec(bl1p.shape, lambda i: (0, 0)),
          pl.BlockSpec(wl2p.shape, lambda i: (0, 0)),
          pl.BlockSpec(bl2p.shape, lambda i: (0, 0)),
      ],
      out_specs=pl.BlockSpec((g, 128), lambda i: (0, 0)),
      out_shape=jax.ShapeDtypeStruct((g, 128), jnp.float32),
      scratch_shapes=[
          pltpu.VMEM((g, 128), jnp.float32),
          pltpu.VMEM((g, 128), jnp.float32),
      ],
  )(a2, hp2, degp, b2r, batch3, wl1p, bl1p, wl2p, bl2p)


def kernel(x, edge_index, batch, W1, b1, W2, b2, Wl1, bl1, Wl2, bl2):
  n0, d_in = x.shape
  e = edge_index.shape[1]
  num_graphs = 64

  n = ((n0 + R - 1) // R) * R
  x = jnp.pad(x, ((0, n - n0), (0, 0)))
  batch = jnp.pad(batch.astype(jnp.int32), (0, n - n0),
                  constant_values=num_graphs)

  ei = edge_index.astype(jnp.int32)
  src, dst = ei[0], ei[1]
  eq = NW * CH * 8
  e_pad = ((e + eq - 1) // eq) * eq
  src_p = jnp.concatenate([src, jnp.zeros((e_pad - e,), jnp.int32)])
  dst_p = jnp.concatenate([dst, jnp.full((e_pad - e,), n, jnp.int32)])
  src2d = src_p.reshape(-1, CH)
  dst2d = dst_p.reshape(-1, CH)
  n_acc = n + NS * L

  degp_full = _deg_call(dst2d, n_acc)
  degp = degp_full[:, :n]

  hp1 = _tc_pre(x, W1, degp)
  a1 = _agg_call(hp1.reshape(2 * n, 128), src2d, dst2d, n, n_acc, 128)
  a1 = a1.reshape(2, n, 128)

  b1r = b1.reshape(1, 256)
  hp2 = _tc_mid(a1, degp, W2, b1r)
  a2 = _agg2_call(hp2, src2d, dst2d, n, n_acc)
  a2 = a2.reshape(2, n, 128)

  b2r = b2.reshape(1, 128)
  batch3 = batch.reshape(n // R, 1, R)
  wl1p = jnp.pad(Wl1, ((0, 0), (0, 64)))
  bl1p = jnp.pad(bl1, (0, 64)).reshape(1, 128)
  wl2p = jnp.pad(Wl2, ((0, 64), (0, 127)))
  bl2p = jnp.pad(bl2, (0, 127)).reshape(1, 128)

  outp = _tc_final(a2, hp2, degp, b2r, batch3, wl1p, bl1p, wl2p, bl2p,
                   num_graphs)
  return outp[:, :1]
